# baseline (speedup 1.0000x reference)
import functools

import jax
import jax.numpy as jnp
from jax import lax
from jax.experimental import pallas as pl
from jax.experimental.pallas import tpu as pltpu
from jax.experimental.pallas import tpu_sc as plsc

N = 10000
E = 320000
D = 128

_NC = 2
_NS = 16
_L = 16
_NW = _NC * _NS
_EPW = E // _NW
_CH = 40
_NCHUNK = _EPW // _CH
_RPT = 624
_TAIL = N - _RPT * _NS


def _gelu(z):
    return 0.5 * z * (1.0 + lax.erf(z * (2.0 ** -0.5)))


def _sc_aggregate(h, x_feat, src, dst, bases):
    mesh = plsc.VectorSubcoreMesh(core_axis_name="c", subcore_axis_name="s")

    @functools.partial(
        pl.kernel,
        mesh=mesh,
        out_type=jax.ShapeDtypeStruct((_NC, N, D), jnp.float32),
        scratch_types=[
            [pltpu.VMEM((_CH,), jnp.int32) for _ in range(6)],
            [pltpu.VMEM((_CH,), jnp.int32) for _ in range(6)],
            [pltpu.VMEM((_CH, D), jnp.float32) for _ in range(3)],
            [pltpu.VMEM((_CH, D), jnp.float32) for _ in range(3)],
            [pltpu.VMEM((_CH, D), jnp.float32) for _ in range(3)],
            pltpu.VMEM_SHARED((N, D), jnp.float32),
            [pltpu.SemaphoreType.DMA for _ in range(6)],
            [pltpu.SemaphoreType.DMA for _ in range(6)],
            [pltpu.SemaphoreType.DMA for _ in range(3)],
            [pltpu.SemaphoreType.DMA for _ in range(3)],
            [pltpu.SemaphoreType.DMA for _ in range(3)],
        ],
    )
    def k(h_hbm, x_hbm, src_hbm, dst_hbm, bases_hbm, out_hbm,
          sidx, didx, hv, bv, mv, acc_sh,
          sem_si, sem_di, sem_h, sem_b, sem_s):
        c = lax.axis_index("c")
        s = lax.axis_index("s")
        wid = s * _NC + c
        r0 = s * _RPT
        pltpu.sync_copy(x_hbm.at[pl.ds(r0, _RPT)], acc_sh.at[pl.ds(r0, _RPT)])

        @pl.when(s == _NS - 1)
        def _():
            pltpu.sync_copy(x_hbm.at[pl.ds(_RPT * _NS, _TAIL)],
                            acc_sh.at[pl.ds(_RPT * _NS, _TAIL)])

        plsc.subcore_barrier()

        ebase = wid * _EPW

        def start_sidx(i, q):
            pltpu.async_copy(src_hbm.at[pl.ds(ebase + i * _CH, _CH)], sidx[q], sem_si[q])

        def start_didx(i, q):
            pltpu.async_copy(dst_hbm.at[pl.ds(ebase + i * _CH, _CH)], didx[q], sem_di[q])

        def start_gather(i, b, q):
            pltpu.async_copy(h_hbm.at[sidx[q]], hv[b], sem_h[b])
            pltpu.async_copy(bases_hbm.at[pl.ds(ebase + i * _CH, _CH)],
                             bv[b], sem_b[b])

        for q in range(6):
            start_sidx(q, q)
        for q in range(3):
            start_didx(q, q)
        for b in range(3):
            pltpu.make_async_copy(src_hbm.at[pl.ds(ebase + b * _CH, _CH)], sidx[b], sem_si[b]).wait()
            start_gather(b, b, b)

        def _maybe(cond, fn):
            if isinstance(cond, bool):
                if cond:
                    fn()
            else:
                pl.when(cond)(fn)

        def chunk(i, b, q, g_drain, g_pre3, g_pre6, unroll):
            q3 = (q + 3) % 6
            pltpu.make_async_copy(h_hbm.at[sidx[q]], hv[b],
                                  sem_h[b]).wait()
            pltpu.make_async_copy(
                bases_hbm.at[pl.ds(ebase + i * _CH, _CH)], bv[b],
                sem_b[b]).wait()

            def _drain():
                pltpu.make_async_copy(mv[b], acc_sh.at[didx[q]],
                                      sem_s[b]).wait()

            _maybe(g_drain, _drain)

            _maybe(g_pre3, lambda: start_didx(i + 3, q3))

            @plsc.parallel_loop(0, _CH, 1, unroll=unroll)
            def _mul(e):
                for j in range(D // _L):
                    sl = pl.ds(j * _L, _L)
                    mv[b][e, sl] = hv[b][e, sl] * bv[b][e, sl]

            pltpu.make_async_copy(dst_hbm.at[pl.ds(ebase + i * _CH, _CH)],
                                  didx[q], sem_di[q]).wait()
            pltpu.async_copy(mv[b], acc_sh.at[didx[q]], sem_s[b], add=True)

            _maybe(g_pre6, lambda: start_sidx(i + 6, q))

            def _next_gather():
                pltpu.make_async_copy(
                    src_hbm.at[pl.ds(ebase + (i + 3) * _CH, _CH)], sidx[q3],
                    sem_si[q3]).wait()
                start_gather(i + 3, b, q3)

            _maybe(g_pre3, _next_gather)

        for i in range(6):
            chunk(i, i % 3, i % 6, i >= 3, True, True, 2)

        def six(i6, carry):
            for kk in range(6):
                chunk(i6 * 6 + kk, kk % 3, kk, True, True, True, 8)
            return carry

        lax.fori_loop(1, _NCHUNK // 6 - 1, six, 0)
        for i in range(_NCHUNK - 10, _NCHUNK):
            chunk(i, i % 3, i % 6, True, i + 3 < _NCHUNK, i + 6 < _NCHUNK, 2)
        for i in range(_NCHUNK - 3, _NCHUNK):
            pltpu.make_async_copy(
                mv[i % 3], acc_sh.at[didx[i % 6]],
                sem_s[i % 3]).wait()
        plsc.subcore_barrier()
        pltpu.sync_copy(acc_sh.at[pl.ds(r0, _RPT)],
                        out_hbm.at[c, pl.ds(r0, _RPT)])

        @pl.when(s == _NS - 1)
        def _():
            pltpu.sync_copy(acc_sh.at[pl.ds(_RPT * _NS, _TAIL)],
                            out_hbm.at[c, pl.ds(_RPT * _NS, _TAIL)])

    return k(h, x_feat, src, dst, bases)


def _tc_preffn(x, W, b):
    def body(x_ref, w_ref, b_ref, o_ref):
        z = jnp.dot(x_ref[...], w_ref[...], preferred_element_type=jnp.float32) + b_ref[...]
        o_ref[...] = _gelu(z)

    return pl.pallas_call(
        body,
        out_shape=jax.ShapeDtypeStruct((N, D), jnp.float32),
    )(x, W, b.reshape(1, D))


def _bn(z, g, b):
    mu = jnp.mean(z, axis=0, keepdims=True)
    var = jnp.mean((z - mu) ** 2, axis=0, keepdims=True)
    return (z - mu) / jnp.sqrt(var + 1e-5) * g + b


def _tc_ffn(x_feat, aggr, W1, b1, g1, be1, W2, b2, g2, be2):
    def body(xf, ag, w1, b1r, g1r, be1r, w2, b2r, g2r, be2r, o_ref):
        x = ag[0] + ag[1] - xf[...]
        y = jnp.dot(x, w1[...], preferred_element_type=jnp.float32) + b1r[...]
        y = _gelu(_bn(y, g1r[...], be1r[...]))
        y = jnp.dot(y, w2[...], preferred_element_type=jnp.float32) + b2r[...]
        y = _gelu(_bn(y, g2r[...], be2r[...]))
        o_ref[...] = x + y

    r = lambda v: v.reshape(1, D)
    return pl.pallas_call(
        body,
        out_shape=jax.ShapeDtypeStruct((N, D), jnp.float32),
    )(x_feat, aggr, W1, r(b1), r(g1), r(be1), W2, r(b2), r(g2), r(be2))


def kernel(x_feat, edge_index, bases, W_pre, b_pre, W1, b1, g1, be1, W2, b2, g2, be2):
    ei = edge_index.astype(jnp.int32)
    src = ei[0]
    dst = ei[1]
    h = _tc_preffn(x_feat, W_pre, b_pre)
    aggr = _sc_aggregate(h, x_feat, src, dst, bases)
    return _tc_ffn(x_feat, aggr, W1, b1, g1, be1, W2, b2, g2, be2)


# speedup vs baseline: 1.0175x; 1.0175x over previous
import functools

import jax
import jax.numpy as jnp
from jax import lax
from jax.experimental import pallas as pl
from jax.experimental.pallas import tpu as pltpu
from jax.experimental.pallas import tpu_sc as plsc

N = 10000
E = 320000
D = 128

_NC = 2
_NS = 16
_L = 16
_NW = _NC * _NS
_EPW = E // _NW
_CH = 40
_NCHUNK = _EPW // _CH
_RPT = 624
_TAIL = N - _RPT * _NS


def _gelu(z):
    return 0.5 * z * (1.0 + lax.erf(z * (2.0 ** -0.5)))


def _sc_aggregate(h, x_feat, src, dst, bases):
    mesh = plsc.VectorSubcoreMesh(core_axis_name="c", subcore_axis_name="s")

    @functools.partial(
        pl.kernel,
        mesh=mesh,
        out_type=jax.ShapeDtypeStruct((_NC, N, D), jnp.float32),
        scratch_types=[
            [pltpu.VMEM((_CH,), jnp.int32) for _ in range(6)],
            [pltpu.VMEM((_CH,), jnp.int32) for _ in range(6)],
            [pltpu.VMEM((_CH, D), jnp.float32) for _ in range(3)],
            [pltpu.VMEM((_CH, D), jnp.float32) for _ in range(3)],
            [pltpu.VMEM((_CH, D), jnp.float32) for _ in range(3)],
            pltpu.VMEM_SHARED((N, D), jnp.float32),
            [pltpu.SemaphoreType.DMA for _ in range(6)],
            [pltpu.SemaphoreType.DMA for _ in range(6)],
            [pltpu.SemaphoreType.DMA for _ in range(3)],
            [pltpu.SemaphoreType.DMA for _ in range(3)],
            [pltpu.SemaphoreType.DMA for _ in range(3)],
        ],
    )
    def k(h_hbm, x_hbm, src_hbm, dst_hbm, bases_hbm, out_hbm,
          sidx, didx, hv, bv, mv, acc_sh,
          sem_si, sem_di, sem_h, sem_b, sem_s):
        c = lax.axis_index("c")
        s = lax.axis_index("s")
        wid = s * _NC + c
        r0 = s * _RPT
        pltpu.sync_copy(x_hbm.at[pl.ds(r0, _RPT)], acc_sh.at[pl.ds(r0, _RPT)])

        @pl.when(s == _NS - 1)
        def _():
            pltpu.sync_copy(x_hbm.at[pl.ds(_RPT * _NS, _TAIL)],
                            acc_sh.at[pl.ds(_RPT * _NS, _TAIL)])

        plsc.subcore_barrier()

        ebase = wid * _EPW

        def start_sidx(i, q):
            pltpu.async_copy(src_hbm.at[pl.ds(ebase + i * _CH, _CH)], sidx[q], sem_si[q])

        def start_didx(i, q):
            pltpu.async_copy(dst_hbm.at[pl.ds(ebase + i * _CH, _CH)], didx[q], sem_di[q])

        def start_gather(i, b, q):
            pltpu.async_copy(h_hbm.at[sidx[q]], hv[b], sem_h[b])
            pltpu.async_copy(bases_hbm.at[pl.ds(ebase + i * _CH, _CH)],
                             bv[b], sem_b[b])

        for q in range(6):
            start_sidx(q, q)
        for q in range(3):
            start_didx(q, q)
        for b in range(3):
            pltpu.make_async_copy(src_hbm.at[pl.ds(ebase + b * _CH, _CH)], sidx[b], sem_si[b]).wait()
            start_gather(b, b, b)

        def chunk(i, b, q):
            q3 = (q + 3) % 6
            pltpu.make_async_copy(h_hbm.at[sidx[q]], hv[b],
                                  sem_h[b]).wait()
            pltpu.make_async_copy(
                bases_hbm.at[pl.ds(ebase + i * _CH, _CH)], bv[b],
                sem_b[b]).wait()

            @pl.when(i >= 3)
            def _():
                pltpu.make_async_copy(mv[b], acc_sh.at[didx[q]],
                                      sem_s[b]).wait()

            @pl.when(i + 3 < _NCHUNK)
            def _():
                start_didx(i + 3, q3)

            @plsc.parallel_loop(0, _CH, 1, unroll=8)
            def _mul(e):
                for j in range(D // _L):
                    sl = pl.ds(j * _L, _L)
                    mv[b][e, sl] = hv[b][e, sl] * bv[b][e, sl]

            pltpu.make_async_copy(dst_hbm.at[pl.ds(ebase + i * _CH, _CH)],
                                  didx[q], sem_di[q]).wait()
            pltpu.async_copy(mv[b], acc_sh.at[didx[q]], sem_s[b], add=True)

            @pl.when(i + 6 < _NCHUNK)
            def _():
                start_sidx(i + 6, q)

            @pl.when(i + 3 < _NCHUNK)
            def _():
                pltpu.make_async_copy(
                    src_hbm.at[pl.ds(ebase + (i + 3) * _CH, _CH)], sidx[q3],
                    sem_si[q3]).wait()
                start_gather(i + 3, b, q3)

        def six(i6, carry):
            for kk in range(6):
                chunk(i6 * 6 + kk, kk % 3, kk)
            return carry

        lax.fori_loop(0, _NCHUNK // 6, six, 0)
        for i in range(_NCHUNK - _NCHUNK % 6, _NCHUNK):
            chunk(i, i % 3, i % 6)
        for i in range(_NCHUNK - 3, _NCHUNK):
            pltpu.make_async_copy(
                mv[i % 3], acc_sh.at[didx[i % 6]],
                sem_s[i % 3]).wait()
        plsc.subcore_barrier()
        pltpu.sync_copy(acc_sh.at[pl.ds(r0, _RPT)],
                        out_hbm.at[c, pl.ds(r0, _RPT)])

        @pl.when(s == _NS - 1)
        def _():
            pltpu.sync_copy(acc_sh.at[pl.ds(_RPT * _NS, _TAIL)],
                            out_hbm.at[c, pl.ds(_RPT * _NS, _TAIL)])

    return k(h, x_feat, src, dst, bases)


def _tc_preffn(x, W, b):
    def body(x_ref, w_ref, b_ref, o_ref):
        z = jnp.dot(x_ref[...], w_ref[...], preferred_element_type=jnp.float32) + b_ref[...]
        o_ref[...] = _gelu(z)

    return pl.pallas_call(
        body,
        out_shape=jax.ShapeDtypeStruct((N, D), jnp.float32),
    )(x, W, b.reshape(1, D))


def _bn(z, g, b):
    mu = jnp.mean(z, axis=0, keepdims=True)
    var = jnp.mean((z - mu) ** 2, axis=0, keepdims=True)
    return (z - mu) / jnp.sqrt(var + 1e-5) * g + b


def _tc_ffn(x_feat, aggr, W1, b1, g1, be1, W2, b2, g2, be2):
    def body(xf, ag, w1, b1r, g1r, be1r, w2, b2r, g2r, be2r, o_ref):
        x = ag[0] + ag[1] - xf[...]
        y = jnp.dot(x, w1[...], preferred_element_type=jnp.float32) + b1r[...]
        y = _gelu(_bn(y, g1r[...], be1r[...]))
        y = jnp.dot(y, w2[...], preferred_element_type=jnp.float32) + b2r[...]
        y = _gelu(_bn(y, g2r[...], be2r[...]))
        o_ref[...] = x + y

    r = lambda v: v.reshape(1, D)
    return pl.pallas_call(
        body,
        out_shape=jax.ShapeDtypeStruct((N, D), jnp.float32),
    )(x_feat, aggr, W1, r(b1), r(g1), r(be1), W2, r(b2), r(g2), r(be2))


def kernel(x_feat, edge_index, bases, W_pre, b_pre, W1, b1, g1, be1, W2, b2, g2, be2):
    ei = edge_index.astype(jnp.int32)
    src = ei[0]
    dst = ei[1]
    h = _tc_preffn(x_feat, W_pre, b_pre)
    aggr = _sc_aggregate(h, x_feat, src, dst, bases)
    return _tc_ffn(x_feat, aggr, W1, b1, g1, be1, W2, b2, g2, be2)
